# Initial kernel scaffold; baseline (speedup 1.0000x reference)
import jax
import jax.numpy as jnp
from jax.experimental import pallas as pl


def kernel(x, edge_index, DDI_features, protein_mask, W1, b1, W2, b2, fc1_W, fc1_b, fc2_W, fc2_b, fc3_W, fc3_b):
    raise NotImplementedError("write your pallas kernel here")


# speedup vs baseline: 26.5603x; 26.5603x over previous
import jax
import jax.numpy as jnp
from jax import lax
from jax.experimental import pallas as pl
from jax.experimental.pallas import tpu as pltpu
from jax.experimental.pallas import tpu_sc as plsc

N_NODES = 10000
N_EDGES = 320000
NPAD = 10240
NC, NS = 2, 16
NW = NC * NS
ROWS_PER_TILE = NPAD // NS
CH = 128
NCHUNK = 79
EPW = NCHUNK * CH
EPAD = NW * EPW


def _sc_mesh():
    return plsc.VectorSubcoreMesh(core_axis_name="c", subcore_axis_name="s",
                                  num_cores=NC, num_subcores=NS)


def _deg_body(dst_hbm, out_hbm, idx_v, ones_v, zrow_v, deg_sh):
    c = lax.axis_index("c")
    s = lax.axis_index("s")
    wid = s * NC + c
    tb = s * ROWS_PER_TILE

    def fill_ones(i, _):
        ones_v[pl.ds(i * 16, 16)] = jnp.ones((16,), jnp.float32)
        return 0
    lax.fori_loop(0, CH // 16, fill_ones, 0)

    def fill_zero(i, _):
        zrow_v[pl.ds(i * 16, 16)] = jnp.zeros((16,), jnp.float32)
        return 0
    lax.fori_loop(0, ROWS_PER_TILE // 16, fill_zero, 0)

    pltpu.sync_copy(dst_hbm.at[wid], idx_v)
    pltpu.sync_copy(zrow_v, deg_sh.at[pl.ds(tb, ROWS_PER_TILE)])
    plsc.subcore_barrier()

    def step(j, _):
        pltpu.sync_copy(ones_v, deg_sh.at[idx_v.at[j]], add=True)
        return 0
    lax.fori_loop(0, NCHUNK, step, 0)

    plsc.subcore_barrier()
    pltpu.sync_copy(deg_sh.at[pl.ds(tb, ROWS_PER_TILE)],
                    out_hbm.at[pl.ds(c * NPAD + tb, ROWS_PER_TILE)])


def _sc_deg(dst_idx):
    f = pl.kernel(
        _deg_body,
        out_type=jax.ShapeDtypeStruct((NC * NPAD,), jnp.float32),
        mesh=_sc_mesh(),
        scratch_types=[
            pltpu.VMEM((NCHUNK, CH), jnp.int32),
            pltpu.VMEM((CH,), jnp.float32),
            pltpu.VMEM((ROWS_PER_TILE,), jnp.float32),
            pltpu.VMEM_SHARED((NPAD,), jnp.float32),
        ],
        name="sc_degree",
    )
    return f(dst_idx)


def _make_scatter_body(W):
    def body(table_hbm, src_hbm, dst_hbm, out_hbm,
             src_v, dst_v, rows_v, zblk_v, acc_sh, gsem):
        c = lax.axis_index("c")
        s = lax.axis_index("s")
        wid = s * NC + c
        tb = s * ROWS_PER_TILE

        for g in range(16):
            for h in range(W // 16):
                zblk_v[g, pl.ds(h * 16, 16)] = jnp.zeros((16,), jnp.float32)

        pltpu.sync_copy(src_hbm.at[wid], src_v)
        pltpu.sync_copy(dst_hbm.at[wid], dst_v)

        def zstep(i, _):
            pltpu.sync_copy(zblk_v, acc_sh.at[pl.ds(tb + i * 16, 16)])
            return 0
        lax.fori_loop(0, ROWS_PER_TILE // 16, zstep, 0)
        plsc.subcore_barrier()

        def step(j, _):
            pltpu.async_copy(table_hbm.at[src_v.at[j]], rows_v, gsem).wait()
            pltpu.sync_copy(rows_v, acc_sh.at[dst_v.at[j]], add=True)
            return 0
        lax.fori_loop(0, NCHUNK, step, 0)

        plsc.subcore_barrier()
        pltpu.sync_copy(acc_sh.at[pl.ds(tb, ROWS_PER_TILE)],
                        out_hbm.at[pl.ds(c * NPAD + tb, ROWS_PER_TILE)])
    return body


def _sc_scatter(table, src_idx, dst_idx, W):
    f = pl.kernel(
        _make_scatter_body(W),
        out_type=jax.ShapeDtypeStruct((NC * NPAD, W), jnp.float32),
        mesh=_sc_mesh(),
        scratch_types=[
            pltpu.VMEM((NCHUNK, CH), jnp.int32),
            pltpu.VMEM((NCHUNK, CH), jnp.int32),
            pltpu.VMEM((CH, W), jnp.float32),
            pltpu.VMEM((16, W), jnp.float32),
            pltpu.VMEM_SHARED((NPAD, W), jnp.float32),
            pltpu.SemaphoreType.DMA,
        ],
        compiler_params=pltpu.CompilerParams(use_tc_tiling_on_sc=False),
        name=f"sc_edge_scatter_w{W}",
    )
    return f(table, src_idx, dst_idx)


_BM = 1024


def _tc_stage1(x_pad, W1, dega, degb):
    def body(x_ref, w_ref, da_ref, db_ref, h_ref, dinv_ref):
        deg = da_ref[...] + db_ref[...] + 1.0
        dinv = lax.rsqrt(deg)
        h = jnp.dot(x_ref[...], w_ref[...], preferred_element_type=jnp.float32)
        h_ref[...] = h * dinv
        dinv_ref[...] = dinv

    return pl.pallas_call(
        body,
        grid=(NPAD // _BM,),
        in_specs=[pl.BlockSpec((_BM, 128), lambda i: (i, 0)),
                  pl.BlockSpec((128, 16), lambda i: (0, 0)),
                  pl.BlockSpec((_BM, 1), lambda i: (i, 0)),
                  pl.BlockSpec((_BM, 1), lambda i: (i, 0))],
        out_specs=[pl.BlockSpec((_BM, 16), lambda i: (i, 0)),
                   pl.BlockSpec((_BM, 1), lambda i: (i, 0))],
        out_shape=[jax.ShapeDtypeStruct((NPAD, 16), jnp.float32),
                   jax.ShapeDtypeStruct((NPAD, 1), jnp.float32)],
        name="tc_stage1",
    )(x_pad, W1, dega, degb)


def _tc_stage2(S1a, S1b, h1p, dinv, b1, W2):
    def body(sa, sb, hp, dv, b_ref, w_ref, out):
        i = pl.program_id(0)
        g = dv[...] * (sa[...] + sb[...] + hp[...]) + b_ref[...]
        g = jnp.maximum(g, 0.0)
        h2 = jnp.dot(g, w_ref[...], preferred_element_type=jnp.float32) * dv[...]
        rows = i * _BM + lax.broadcasted_iota(jnp.int32, (_BM, 1), 0)
        out[...] = jnp.where(rows < N_NODES, h2, 0.0)

    return pl.pallas_call(
        body,
        grid=(NPAD // _BM,),
        in_specs=[pl.BlockSpec((_BM, 16), lambda i: (i, 0)),
                  pl.BlockSpec((_BM, 16), lambda i: (i, 0)),
                  pl.BlockSpec((_BM, 16), lambda i: (i, 0)),
                  pl.BlockSpec((_BM, 1), lambda i: (i, 0)),
                  pl.BlockSpec((1, 16), lambda i: (0, 0)),
                  pl.BlockSpec((16, 32), lambda i: (0, 0))],
        out_specs=pl.BlockSpec((_BM, 32), lambda i: (i, 0)),
        out_shape=jax.ShapeDtypeStruct((NPAD, 32), jnp.float32),
        name="tc_stage2",
    )(S1a, S1b, h1p, dinv, b1, W2)


def _tc_stage3(S2a, S2b, h2p, dinv, b2):
    def body(sa, sb, hp, dv, b_ref, out):
        o = dv[...] * (sa[...] + sb[...] + hp[...]) + b_ref[...]
        out[...] = jnp.maximum(o, 0.0)

    return pl.pallas_call(
        body,
        grid=(NPAD // _BM,),
        in_specs=[pl.BlockSpec((_BM, 32), lambda i: (i, 0)),
                  pl.BlockSpec((_BM, 32), lambda i: (i, 0)),
                  pl.BlockSpec((_BM, 32), lambda i: (i, 0)),
                  pl.BlockSpec((_BM, 1), lambda i: (i, 0)),
                  pl.BlockSpec((1, 32), lambda i: (0, 0))],
        out_specs=pl.BlockSpec((_BM, 32), lambda i: (i, 0)),
        out_shape=jax.ShapeDtypeStruct((NPAD, 32), jnp.float32),
        name="tc_stage3",
    )(S2a, S2b, h2p, dinv, b2)


def _tc_ddi(DDI, w1, b1, w2, b2, w3, b3):
    def body(x_ref, w1r, b1r, w2r, b2r, w3r, b3r, out):
        d = jnp.dot(x_ref[...], w1r[...], preferred_element_type=jnp.float32)
        d = jnp.maximum(d + b1r[...], 0.0)
        d = jnp.dot(d, w2r[...], preferred_element_type=jnp.float32)
        d = jnp.maximum(d + b2r[...], 0.0)
        d = jnp.dot(d, w3r[...], preferred_element_type=jnp.float32)
        out[...] = jnp.maximum(d + b3r[...], 0.0)

    return pl.pallas_call(
        body,
        out_shape=jax.ShapeDtypeStruct((DDI.shape[0], 1), jnp.float32),
        name="tc_ddi",
    )(DDI, w1, b1, w2, b2, w3, b3)


def kernel(x, edge_index, DDI_features, protein_mask, W1, b1, W2, b2,
           fc1_W, fc1_b, fc2_W, fc2_b, fc3_W, fc3_b):
    src = edge_index[0].astype(jnp.int32)
    dst = edge_index[1].astype(jnp.int32)
    pad = jnp.full((EPAD - N_EDGES,), N_NODES, jnp.int32)
    src_p = jnp.concatenate([src, pad]).reshape(NW, NCHUNK, CH)
    dst_p = jnp.concatenate([dst, pad]).reshape(NW, NCHUNK, CH)
    x_pad = jnp.pad(x, ((0, NPAD - N_NODES), (0, 0)))

    deg2 = _sc_deg(dst_p)
    dega = deg2[:NPAD].reshape(NPAD, 1)
    degb = deg2[NPAD:].reshape(NPAD, 1)

    h1p, dinv = _tc_stage1(x_pad, W1, dega, degb)
    S1 = _sc_scatter(h1p, src_p, dst_p, 16)
    h2p = _tc_stage2(S1[:NPAD], S1[NPAD:], h1p, dinv,
                     b1.reshape(1, 16), W2)
    S2 = _sc_scatter(h2p, src_p, dst_p, 32)
    ppi = _tc_stage3(S2[:NPAD], S2[NPAD:], h2p, dinv, b2.reshape(1, 32))
    PPI_x = ppi[:N_NODES]

    DDI_x = _tc_ddi(DDI_features, fc1_W, fc1_b.reshape(1, 64),
                    fc2_W, fc2_b.reshape(1, 16), fc3_W, fc3_b.reshape(1, 1))
    return (PPI_x, DDI_x)


# speedup vs baseline: 31.0993x; 1.1709x over previous
import jax
import jax.numpy as jnp
from jax import lax
from jax.experimental import pallas as pl
from jax.experimental.pallas import tpu as pltpu
from jax.experimental.pallas import tpu_sc as plsc

N_NODES = 10000
N_EDGES = 320000
NPAD = 10240
NC, NS = 2, 16
NW = NC * NS
ROWS_PER_TILE = NPAD // NS
CH = 128
NCHUNK = 80
NBUF = 4
EPW = NCHUNK * CH
EPAD = NW * EPW


def _sc_mesh():
    return plsc.VectorSubcoreMesh(core_axis_name="c", subcore_axis_name="s",
                                  num_cores=NC, num_subcores=NS)


def _deg_body(dst_hbm, out_hbm, idx_v, ones_v, zrow_v, deg_sh):
    c = lax.axis_index("c")
    s = lax.axis_index("s")
    wid = s * NC + c
    tb = s * ROWS_PER_TILE

    def fill_ones(i, _):
        ones_v[pl.ds(i * 16, 16)] = jnp.ones((16,), jnp.float32)
        return 0
    lax.fori_loop(0, CH // 16, fill_ones, 0)

    def fill_zero(i, _):
        zrow_v[pl.ds(i * 16, 16)] = jnp.zeros((16,), jnp.float32)
        return 0
    lax.fori_loop(0, ROWS_PER_TILE // 16, fill_zero, 0)

    pltpu.sync_copy(dst_hbm.at[wid], idx_v)
    pltpu.sync_copy(zrow_v, deg_sh.at[pl.ds(tb, ROWS_PER_TILE)])
    plsc.subcore_barrier()

    def step(j, _):
        pltpu.sync_copy(ones_v, deg_sh.at[idx_v.at[j]], add=True)
        return 0
    lax.fori_loop(0, NCHUNK, step, 0)

    plsc.subcore_barrier()
    pltpu.sync_copy(deg_sh.at[pl.ds(tb, ROWS_PER_TILE)],
                    out_hbm.at[pl.ds(c * NPAD + tb, ROWS_PER_TILE)])


def _sc_deg(dst_idx):
    f = pl.kernel(
        _deg_body,
        out_type=jax.ShapeDtypeStruct((NC * NPAD,), jnp.float32),
        mesh=_sc_mesh(),
        scratch_types=[
            pltpu.VMEM((NCHUNK, CH), jnp.int32),
            pltpu.VMEM((CH,), jnp.float32),
            pltpu.VMEM((ROWS_PER_TILE,), jnp.float32),
            pltpu.VMEM_SHARED((NPAD,), jnp.float32),
        ],
        name="sc_degree",
    )
    return f(dst_idx)


def _make_scatter_body(W):
    def body(table_hbm, src_hbm, dst_hbm, out_hbm,
             src_v, dst_v, rows_v, zblk_v, acc_sh,
             g0, g1, g2, g3, s0, s1, s2, s3):
        gsems = (g0, g1, g2, g3)
        ssems = (s0, s1, s2, s3)
        c = lax.axis_index("c")
        s = lax.axis_index("s")
        wid = s * NC + c
        tb = s * ROWS_PER_TILE

        for g in range(16):
            for h in range(W // 16):
                zblk_v[g, pl.ds(h * 16, 16)] = jnp.zeros((16,), jnp.float32)

        pltpu.sync_copy(src_hbm.at[wid], src_v)
        pltpu.sync_copy(dst_hbm.at[wid], dst_v)

        def zstep(i, _):
            pltpu.sync_copy(zblk_v, acc_sh.at[pl.ds(tb + i * 16, 16)])
            return 0
        lax.fori_loop(0, ROWS_PER_TILE // 16, zstep, 0)
        plsc.subcore_barrier()

        def gstart(j, b):
            pltpu.async_copy(table_hbm.at[src_v.at[j]], rows_v.at[b], gsems[b])

        def gwait(j, b):
            pltpu.make_async_copy(table_hbm.at[src_v.at[j]], rows_v.at[b],
                                  gsems[b]).wait()

        def sstart(j, b):
            pltpu.async_copy(rows_v.at[b], acc_sh.at[dst_v.at[j]], ssems[b],
                             add=True)

        def swait(j, b):
            pltpu.make_async_copy(rows_v.at[b], acc_sh.at[dst_v.at[j]],
                                  ssems[b]).wait()

        gstart(0, 0)
        gstart(1, 1)
        gstart(2, 2)

        def step(g, _):
            for b in range(NBUF):
                j = NBUF * g + b
                gwait(j, b)
                sstart(j, b)
                nb = (b + 3) % NBUF
                nxt = j + 3
                if b == 0:
                    @pl.when(g == 0)
                    def _():
                        gstart(3, 3)

                    @pl.when(g > 0)
                    def _():
                        swait(j - 1, nb)
                        gstart(nxt, nb)
                else:
                    @pl.when(nxt < NCHUNK)
                    def _():
                        swait(j - 1, nb)
                        gstart(nxt, nb)
            return 0
        lax.fori_loop(0, NCHUNK // NBUF, step, 0)

        for b in range(NBUF):
            swait(NCHUNK - NBUF + b, b)

        plsc.subcore_barrier()
        pltpu.sync_copy(acc_sh.at[pl.ds(tb, ROWS_PER_TILE)],
                        out_hbm.at[pl.ds(c * NPAD + tb, ROWS_PER_TILE)])
    return body


def _sc_scatter(table, src_idx, dst_idx, W):
    f = pl.kernel(
        _make_scatter_body(W),
        out_type=jax.ShapeDtypeStruct((NC * NPAD, W), jnp.float32),
        mesh=_sc_mesh(),
        scratch_types=[
            pltpu.VMEM((NCHUNK, CH), jnp.int32),
            pltpu.VMEM((NCHUNK, CH), jnp.int32),
            pltpu.VMEM((NBUF, CH, W), jnp.float32),
            pltpu.VMEM((16, W), jnp.float32),
            pltpu.VMEM_SHARED((NPAD, W), jnp.float32),
            pltpu.SemaphoreType.DMA,
            pltpu.SemaphoreType.DMA,
            pltpu.SemaphoreType.DMA,
            pltpu.SemaphoreType.DMA,
            pltpu.SemaphoreType.DMA,
            pltpu.SemaphoreType.DMA,
            pltpu.SemaphoreType.DMA,
            pltpu.SemaphoreType.DMA,
        ],
        compiler_params=pltpu.CompilerParams(use_tc_tiling_on_sc=False),
        name=f"sc_edge_scatter_w{W}",
    )
    return f(table, src_idx, dst_idx)


_BM = 1024


def _tc_stage1(x_pad, W1, dega, degb):
    def body(x_ref, w_ref, da_ref, db_ref, h_ref, dinv_ref):
        deg = da_ref[...] + db_ref[...] + 1.0
        dinv = lax.rsqrt(deg)
        h = jnp.dot(x_ref[...], w_ref[...], preferred_element_type=jnp.float32)
        h_ref[...] = h * dinv
        dinv_ref[...] = dinv

    return pl.pallas_call(
        body,
        grid=(NPAD // _BM,),
        in_specs=[pl.BlockSpec((_BM, 128), lambda i: (i, 0)),
                  pl.BlockSpec((128, 16), lambda i: (0, 0)),
                  pl.BlockSpec((_BM, 1), lambda i: (i, 0)),
                  pl.BlockSpec((_BM, 1), lambda i: (i, 0))],
        out_specs=[pl.BlockSpec((_BM, 16), lambda i: (i, 0)),
                   pl.BlockSpec((_BM, 1), lambda i: (i, 0))],
        out_shape=[jax.ShapeDtypeStruct((NPAD, 16), jnp.float32),
                   jax.ShapeDtypeStruct((NPAD, 1), jnp.float32)],
        name="tc_stage1",
    )(x_pad, W1, dega, degb)


def _tc_stage2(S1a, S1b, h1p, dinv, b1, W2):
    def body(sa, sb, hp, dv, b_ref, w_ref, out):
        i = pl.program_id(0)
        g = dv[...] * (sa[...] + sb[...] + hp[...]) + b_ref[...]
        g = jnp.maximum(g, 0.0)
        h2 = jnp.dot(g, w_ref[...], preferred_element_type=jnp.float32) * dv[...]
        rows = i * _BM + lax.broadcasted_iota(jnp.int32, (_BM, 1), 0)
        out[...] = jnp.where(rows < N_NODES, h2, 0.0)

    return pl.pallas_call(
        body,
        grid=(NPAD // _BM,),
        in_specs=[pl.BlockSpec((_BM, 16), lambda i: (i, 0)),
                  pl.BlockSpec((_BM, 16), lambda i: (i, 0)),
                  pl.BlockSpec((_BM, 16), lambda i: (i, 0)),
                  pl.BlockSpec((_BM, 1), lambda i: (i, 0)),
                  pl.BlockSpec((1, 16), lambda i: (0, 0)),
                  pl.BlockSpec((16, 32), lambda i: (0, 0))],
        out_specs=pl.BlockSpec((_BM, 32), lambda i: (i, 0)),
        out_shape=jax.ShapeDtypeStruct((NPAD, 32), jnp.float32),
        name="tc_stage2",
    )(S1a, S1b, h1p, dinv, b1, W2)


def _tc_stage3(S2a, S2b, h2p, dinv, b2):
    def body(sa, sb, hp, dv, b_ref, out):
        o = dv[...] * (sa[...] + sb[...] + hp[...]) + b_ref[...]
        out[...] = jnp.maximum(o, 0.0)

    return pl.pallas_call(
        body,
        grid=(NPAD // _BM,),
        in_specs=[pl.BlockSpec((_BM, 32), lambda i: (i, 0)),
                  pl.BlockSpec((_BM, 32), lambda i: (i, 0)),
                  pl.BlockSpec((_BM, 32), lambda i: (i, 0)),
                  pl.BlockSpec((_BM, 1), lambda i: (i, 0)),
                  pl.BlockSpec((1, 32), lambda i: (0, 0))],
        out_specs=pl.BlockSpec((_BM, 32), lambda i: (i, 0)),
        out_shape=jax.ShapeDtypeStruct((NPAD, 32), jnp.float32),
        name="tc_stage3",
    )(S2a, S2b, h2p, dinv, b2)


def _tc_ddi(DDI, w1, b1, w2, b2, w3, b3):
    def body(x_ref, w1r, b1r, w2r, b2r, w3r, b3r, out):
        d = jnp.dot(x_ref[...], w1r[...], preferred_element_type=jnp.float32)
        d = jnp.maximum(d + b1r[...], 0.0)
        d = jnp.dot(d, w2r[...], preferred_element_type=jnp.float32)
        d = jnp.maximum(d + b2r[...], 0.0)
        d = jnp.dot(d, w3r[...], preferred_element_type=jnp.float32)
        out[...] = jnp.maximum(d + b3r[...], 0.0)

    return pl.pallas_call(
        body,
        out_shape=jax.ShapeDtypeStruct((DDI.shape[0], 1), jnp.float32),
        name="tc_ddi",
    )(DDI, w1, b1, w2, b2, w3, b3)


def kernel(x, edge_index, DDI_features, protein_mask, W1, b1, W2, b2,
           fc1_W, fc1_b, fc2_W, fc2_b, fc3_W, fc3_b):
    src = edge_index[0].astype(jnp.int32)
    dst = edge_index[1].astype(jnp.int32)
    pad = jnp.full((EPAD - N_EDGES,), N_NODES, jnp.int32)
    src_p = jnp.concatenate([src, pad]).reshape(NW, NCHUNK, CH)
    dst_p = jnp.concatenate([dst, pad]).reshape(NW, NCHUNK, CH)
    x_pad = jnp.pad(x, ((0, NPAD - N_NODES), (0, 0)))

    deg2 = _sc_deg(dst_p)
    dega = deg2[:NPAD].reshape(NPAD, 1)
    degb = deg2[NPAD:].reshape(NPAD, 1)

    h1p, dinv = _tc_stage1(x_pad, W1, dega, degb)
    S1 = _sc_scatter(h1p, src_p, dst_p, 16)
    h2p = _tc_stage2(S1[:NPAD], S1[NPAD:], h1p, dinv,
                     b1.reshape(1, 16), W2)
    S2 = _sc_scatter(h2p, src_p, dst_p, 32)
    ppi = _tc_stage3(S2[:NPAD], S2[NPAD:], h2p, dinv, b2.reshape(1, 32))
    PPI_x = ppi[:N_NODES]

    DDI_x = _tc_ddi(DDI_features, fc1_W, fc1_b.reshape(1, 64),
                    fc2_W, fc2_b.reshape(1, 16), fc3_W, fc3_b.reshape(1, 1))
    return (PPI_x, DDI_x)


# speedup vs baseline: 34.0658x; 1.0954x over previous
import jax
import jax.numpy as jnp
from jax import lax
from jax.experimental import pallas as pl
from jax.experimental.pallas import tpu as pltpu
from jax.experimental.pallas import tpu_sc as plsc

N_NODES = 10000
N_EDGES = 320000
NPAD = 10240
NC, NS = 2, 16
NW = NC * NS
ROWS_PER_TILE = NPAD // NS
CH = 128
TCH = 2560
NBUF = 4

DEG_SPLIT = (88, 72)
W16_SPLIT = (104, 56)
W32_SPLIT = (112, 48)


def _sc_mesh():
    return plsc.VectorSubcoreMesh(core_axis_name="c", subcore_axis_name="s",
                                  num_cores=NC, num_subcores=NS)


def _make_deg_body(split):
    A, B = split

    def body(edge_hbm, out_hbm, idx_v, ones_v, zrow_v, deg_sh):
        c = lax.axis_index("c")
        s = lax.axis_index("s")
        tb = s * ROWS_PER_TILE
        nch = jnp.where(c == 0, A, B)

        def fill_ones(i, _):
            ones_v[pl.ds(i * 16, 16)] = jnp.ones((16,), jnp.float32)
            return 0
        lax.fori_loop(0, CH // 16, fill_ones, 0)

        def fill_zero(i, _):
            zrow_v[pl.ds(i * 16, 16)] = jnp.zeros((16,), jnp.float32)
            return 0
        lax.fori_loop(0, ROWS_PER_TILE // 16, fill_zero, 0)

        @pl.when(c == 0)
        def _():
            pltpu.sync_copy(edge_hbm.at[1].at[pl.ds(s * A, A)],
                            idx_v.at[pl.ds(0, A)])

        @pl.when(c == 1)
        def _():
            pltpu.sync_copy(edge_hbm.at[1].at[pl.ds(NS * A + s * B, B)],
                            idx_v.at[pl.ds(0, B)])

        pltpu.sync_copy(zrow_v, deg_sh.at[pl.ds(tb, ROWS_PER_TILE)])
        plsc.subcore_barrier()

        def step(j, _):
            pltpu.sync_copy(ones_v, deg_sh.at[idx_v.at[j]], add=True)
            return 0
        lax.fori_loop(0, nch, step, 0)

        plsc.subcore_barrier()
        pltpu.sync_copy(deg_sh.at[pl.ds(tb, ROWS_PER_TILE)],
                        out_hbm.at[pl.ds(c * NPAD + tb, ROWS_PER_TILE)])
    return body


def _sc_deg(edges):
    A = DEG_SPLIT[0]
    f = pl.kernel(
        _make_deg_body(DEG_SPLIT),
        out_type=jax.ShapeDtypeStruct((NC * NPAD,), jnp.float32),
        mesh=_sc_mesh(),
        scratch_types=[
            pltpu.VMEM((A, CH), jnp.int32),
            pltpu.VMEM((CH,), jnp.float32),
            pltpu.VMEM((ROWS_PER_TILE,), jnp.float32),
            pltpu.VMEM_SHARED((NPAD,), jnp.float32),
        ],
        name="sc_degree",
    )
    return f(edges)


def _make_scatter_body(W, split):
    A, B = split

    def body(table_hbm, edge_hbm, out_hbm,
             src_v, dst_v, rows_v, zblk_v, acc_sh,
             g0, g1, g2, g3, s0, s1, s2, s3):
        gsems = (g0, g1, g2, g3)
        ssems = (s0, s1, s2, s3)
        c = lax.axis_index("c")
        s = lax.axis_index("s")
        tb = s * ROWS_PER_TILE
        nch = jnp.where(c == 0, A, B)

        for g in range(16):
            for h in range(W // 16):
                zblk_v[g, pl.ds(h * 16, 16)] = jnp.zeros((16,), jnp.float32)

        @pl.when(c == 0)
        def _():
            pltpu.sync_copy(edge_hbm.at[0].at[pl.ds(s * A, A)],
                            src_v.at[pl.ds(0, A)])
            pltpu.sync_copy(edge_hbm.at[1].at[pl.ds(s * A, A)],
                            dst_v.at[pl.ds(0, A)])

        @pl.when(c == 1)
        def _():
            pltpu.sync_copy(edge_hbm.at[0].at[pl.ds(NS * A + s * B, B)],
                            src_v.at[pl.ds(0, B)])
            pltpu.sync_copy(edge_hbm.at[1].at[pl.ds(NS * A + s * B, B)],
                            dst_v.at[pl.ds(0, B)])

        def zstep(i, _):
            pltpu.sync_copy(zblk_v, acc_sh.at[pl.ds(tb + i * 16, 16)])
            return 0
        lax.fori_loop(0, ROWS_PER_TILE // 16, zstep, 0)
        plsc.subcore_barrier()

        def gstart(j, b):
            pltpu.async_copy(table_hbm.at[src_v.at[j]], rows_v.at[b], gsems[b])

        def gwait(j, b):
            pltpu.make_async_copy(table_hbm.at[src_v.at[j]], rows_v.at[b],
                                  gsems[b]).wait()

        def sstart(j, b):
            pltpu.async_copy(rows_v.at[b], acc_sh.at[dst_v.at[j]], ssems[b],
                             add=True)

        def swait(j, b):
            pltpu.make_async_copy(rows_v.at[b], acc_sh.at[dst_v.at[j]],
                                  ssems[b]).wait()

        gstart(0, 0)
        gstart(1, 1)
        gstart(2, 2)

        def step(g, _):
            for b in range(NBUF):
                j = NBUF * g + b
                gwait(j, b)
                sstart(j, b)
                nb = (b + 3) % NBUF
                nxt = j + 3

                @pl.when((nxt < nch) & (j >= 1))
                def _():
                    swait(j - 1, nb)
                    gstart(nxt, nb)

                if b == 0:
                    @pl.when((g == 0) & (3 < nch))
                    def _():
                        gstart(3, 3)
            return 0
        lax.fori_loop(0, nch // NBUF, step, 0)

        for b in range(NBUF):
            swait(nch - NBUF + b, b)

        plsc.subcore_barrier()
        pltpu.sync_copy(acc_sh.at[pl.ds(tb, ROWS_PER_TILE)],
                        out_hbm.at[pl.ds(c * NPAD + tb, ROWS_PER_TILE)])
    return body


def _sc_scatter(table, edges, W, split):
    A = split[0]
    f = pl.kernel(
        _make_scatter_body(W, split),
        out_type=jax.ShapeDtypeStruct((NC * NPAD, W), jnp.float32),
        mesh=_sc_mesh(),
        scratch_types=[
            pltpu.VMEM((A, CH), jnp.int32),
            pltpu.VMEM((A, CH), jnp.int32),
            pltpu.VMEM((NBUF, CH, W), jnp.float32),
            pltpu.VMEM((16, W), jnp.float32),
            pltpu.VMEM_SHARED((NPAD, W), jnp.float32),
            pltpu.SemaphoreType.DMA,
            pltpu.SemaphoreType.DMA,
            pltpu.SemaphoreType.DMA,
            pltpu.SemaphoreType.DMA,
            pltpu.SemaphoreType.DMA,
            pltpu.SemaphoreType.DMA,
            pltpu.SemaphoreType.DMA,
            pltpu.SemaphoreType.DMA,
        ],
        compiler_params=pltpu.CompilerParams(use_tc_tiling_on_sc=False),
        name=f"sc_edge_scatter_w{W}",
    )
    return f(table, edges)


_BM = 1024


def _tc_stage1(x_pad, W1, dega, degb):
    def body(x_ref, w_ref, da_ref, db_ref, h_ref, dinv_ref):
        deg = da_ref[...] + db_ref[...] + 1.0
        dinv = lax.rsqrt(deg)
        h = jnp.dot(x_ref[...], w_ref[...], preferred_element_type=jnp.float32)
        h_ref[...] = h * dinv
        dinv_ref[...] = dinv

    return pl.pallas_call(
        body,
        grid=(NPAD // _BM,),
        in_specs=[pl.BlockSpec((_BM, 128), lambda i: (i, 0)),
                  pl.BlockSpec((128, 16), lambda i: (0, 0)),
                  pl.BlockSpec((_BM, 1), lambda i: (i, 0)),
                  pl.BlockSpec((_BM, 1), lambda i: (i, 0))],
        out_specs=[pl.BlockSpec((_BM, 16), lambda i: (i, 0)),
                   pl.BlockSpec((_BM, 1), lambda i: (i, 0))],
        out_shape=[jax.ShapeDtypeStruct((NPAD, 16), jnp.float32),
                   jax.ShapeDtypeStruct((NPAD, 1), jnp.float32)],
        name="tc_stage1",
    )(x_pad, W1, dega, degb)


def _tc_stage2(S1a, S1b, h1p, dinv, b1, W2):
    def body(sa, sb, hp, dv, b_ref, w_ref, out):
        i = pl.program_id(0)
        g = dv[...] * (sa[...] + sb[...] + hp[...]) + b_ref[...]
        g = jnp.maximum(g, 0.0)
        h2 = jnp.dot(g, w_ref[...], preferred_element_type=jnp.float32) * dv[...]
        rows = i * _BM + lax.broadcasted_iota(jnp.int32, (_BM, 1), 0)
        out[...] = jnp.where(rows < N_NODES, h2, 0.0)

    return pl.pallas_call(
        body,
        grid=(NPAD // _BM,),
        in_specs=[pl.BlockSpec((_BM, 16), lambda i: (i, 0)),
                  pl.BlockSpec((_BM, 16), lambda i: (i, 0)),
                  pl.BlockSpec((_BM, 16), lambda i: (i, 0)),
                  pl.BlockSpec((_BM, 1), lambda i: (i, 0)),
                  pl.BlockSpec((1, 16), lambda i: (0, 0)),
                  pl.BlockSpec((16, 32), lambda i: (0, 0))],
        out_specs=pl.BlockSpec((_BM, 32), lambda i: (i, 0)),
        out_shape=jax.ShapeDtypeStruct((NPAD, 32), jnp.float32),
        name="tc_stage2",
    )(S1a, S1b, h1p, dinv, b1, W2)


def _tc_stage3(S2a, S2b, h2p, dinv, b2):
    def body(sa, sb, hp, dv, b_ref, out):
        o = dv[...] * (sa[...] + sb[...] + hp[...]) + b_ref[...]
        out[...] = jnp.maximum(o, 0.0)

    return pl.pallas_call(
        body,
        grid=(NPAD // _BM,),
        in_specs=[pl.BlockSpec((_BM, 32), lambda i: (i, 0)),
                  pl.BlockSpec((_BM, 32), lambda i: (i, 0)),
                  pl.BlockSpec((_BM, 32), lambda i: (i, 0)),
                  pl.BlockSpec((_BM, 1), lambda i: (i, 0)),
                  pl.BlockSpec((1, 32), lambda i: (0, 0))],
        out_specs=pl.BlockSpec((_BM, 32), lambda i: (i, 0)),
        out_shape=jax.ShapeDtypeStruct((NPAD, 32), jnp.float32),
        name="tc_stage3",
    )(S2a, S2b, h2p, dinv, b2)


def _tc_ddi(DDI, w1, b1, w2, b2, w3, b3):
    def body(x_ref, w1r, b1r, w2r, b2r, w3r, b3r, out):
        d = jnp.dot(x_ref[...], w1r[...], preferred_element_type=jnp.float32)
        d = jnp.maximum(d + b1r[...], 0.0)
        d = jnp.dot(d, w2r[...], preferred_element_type=jnp.float32)
        d = jnp.maximum(d + b2r[...], 0.0)
        d = jnp.dot(d, w3r[...], preferred_element_type=jnp.float32)
        out[...] = jnp.maximum(d + b3r[...], 0.0)

    return pl.pallas_call(
        body,
        out_shape=jax.ShapeDtypeStruct((DDI.shape[0], 1), jnp.float32),
        name="tc_ddi",
    )(DDI, w1, b1, w2, b2, w3, b3)


def kernel(x, edge_index, DDI_features, protein_mask, W1, b1, W2, b2,
           fc1_W, fc1_b, fc2_W, fc2_b, fc3_W, fc3_b):
    edges = jnp.pad(edge_index.astype(jnp.int32),
                    ((0, 0), (0, TCH * CH - N_EDGES)),
                    constant_values=N_NODES).reshape(2, TCH, CH)
    x_pad = jnp.pad(x, ((0, NPAD - N_NODES), (0, 0)))

    deg2 = _sc_deg(edges)
    dega = deg2[:NPAD].reshape(NPAD, 1)
    degb = deg2[NPAD:].reshape(NPAD, 1)

    h1p, dinv = _tc_stage1(x_pad, W1, dega, degb)
    S1 = _sc_scatter(h1p, edges, 16, W16_SPLIT)
    h2p = _tc_stage2(S1[:NPAD], S1[NPAD:], h1p, dinv,
                     b1.reshape(1, 16), W2)
    S2 = _sc_scatter(h2p, edges, 32, W32_SPLIT)
    ppi = _tc_stage3(S2[:NPAD], S2[NPAD:], h2p, dinv, b2.reshape(1, 32))
    PPI_x = ppi[:N_NODES]

    DDI_x = _tc_ddi(DDI_features, fc1_W, fc1_b.reshape(1, 64),
                    fc2_W, fc2_b.reshape(1, 16), fc3_W, fc3_b.reshape(1, 1))
    return (PPI_x, DDI_x)
